# baseline (speedup 1.0000x reference)
import jax
import jax.numpy as jnp
from jax.experimental import pallas as pl

_B = 4096
_L = 200
_D = 32
_BLK = 1024


def _body(labels_ref, w_ref, out_ref):
    labT = labels_ref[...].T
    m = jax.lax.broadcast_in_dim(labT != 0, (_L, _D, _BLK), (0, 2))
    w = w_ref[...]
    out_ref[...] = jnp.where(m, w, 0.0)


def kernel(labels, weight):
    w3 = jax.lax.slice(weight, (1, 0), (1 + _L, _D)).reshape(_L, _D, 1)
    outT = pl.pallas_call(
        _body,
        grid=(_B // _BLK,),
        in_specs=[
            pl.BlockSpec((_BLK, _L), lambda i: (i, 0)),
            pl.BlockSpec((_L, _D, 1), lambda i: (0, 0, 0)),
        ],
        out_specs=pl.BlockSpec((_L, _D, _BLK), lambda i: (0, 0, i)),
        out_shape=jax.ShapeDtypeStruct((_L, _D, _B), jnp.float32),
    )(labels, w3)
    return outT.transpose(2, 0, 1)


# speedup vs baseline: 1.0518x; 1.0518x over previous
import jax
import jax.numpy as jnp
from jax.experimental import pallas as pl

_B = 4096
_L = 200
_D = 32
_BLK = 256


def _body(labels_ref, w_ref, out_ref):
    labT = labels_ref[...].T
    m = jax.lax.broadcast_in_dim(labT != 0, (_L, _D, _BLK), (0, 2))
    w = w_ref[...]
    out_ref[...] = jnp.where(m, w, 0.0)


def kernel(labels, weight):
    w3 = jax.lax.slice(weight, (1, 0), (1 + _L, _D)).reshape(_L, _D, 1)
    outT = pl.pallas_call(
        _body,
        grid=(_B // _BLK,),
        in_specs=[
            pl.BlockSpec((_BLK, _L), lambda i: (i, 0)),
            pl.BlockSpec((_L, _D, 1), lambda i: (0, 0, 0)),
        ],
        out_specs=pl.BlockSpec((_L, _D, _BLK), lambda i: (0, 0, i)),
        out_shape=jax.ShapeDtypeStruct((_L, _D, _B), jnp.float32),
    )(labels, w3)
    return outT.transpose(2, 0, 1)
